# Initial kernel scaffold; baseline (speedup 1.0000x reference)
import jax
import jax.numpy as jnp
from jax.experimental import pallas as pl


def kernel(L_indices, L_values, mask, inputs, W1, b1, rn_gamma, rn_beta, rn_W, rn_b, g2, be2, W2, b2):
    raise NotImplementedError("write your pallas kernel here")


# speedup vs baseline: 2.0498x; 2.0498x over previous
import functools

import jax
import jax.numpy as jnp
from jax import lax
from jax.experimental import pallas as pl
from jax.experimental.pallas import tpu as pltpu
from jax.experimental.pallas import tpu_sc as plsc

N = 10000
D = 128
E = 320000
LAYERS = 6

NC = 2
NS = 16
NW = NC * NS
K = 128
NCH = 80
EPW = K * NCH
E_PAD = NW * EPW
RPW = 624
TAIL = N - NS * RPW
SH = 14
MSK = (1 << SH) - 1

_f32 = jnp.float32
_i32 = jnp.int32


def _spmm_body(h_hbm, pk_hbm, vals_hbm, out_hbm,
               pk_v, gidx, sidx, vbuf, gbuf, acc_sh, semg, sems):
    c = lax.axis_index("c")
    s = lax.axis_index("s")
    wid = s * NC + c

    def _zr(i, carry):
        gbuf[i // 8, pl.ds((i % 8) * 16, 16)] = jnp.zeros((16,), _f32)
        return carry

    lax.fori_loop(0, K * D // 16, _zr, 0)
    for z in range(0, RPW, K):
        cnt = min(K, RPW - z)
        pltpu.sync_copy(gbuf.at[pl.ds(0, cnt)], acc_sh.at[pl.ds(s * RPW + z, cnt)])

    @pl.when(s == NS - 1)
    def _zero_tail():
        pltpu.sync_copy(gbuf.at[pl.ds(0, TAIL)], acc_sh.at[pl.ds(NS * RPW, TAIL)])

    plsc.subcore_barrier()

    pltpu.sync_copy(pk_hbm.at[wid], pk_v)

    def _chunk(j, carry):
        pltpu.sync_copy(vals_hbm.at[wid, j], vbuf)
        for g in range(K // 16):
            sl = pl.ds(g * 16, 16)
            pk = pk_v[j, sl]
            gidx[sl] = pk & MSK
            sidx[sl] = lax.shift_right_logical(pk, SH)
        pltpu.async_copy(h_hbm.at[gidx], gbuf, semg).wait()

        def _scale(e, carry2):
            vv = vbuf[e]
            for dd in range(D // 16):
                sl = pl.ds(dd * 16, 16)
                gbuf[e, sl] = gbuf[e, sl] * vv
            return carry2

        lax.fori_loop(0, K, _scale, 0)
        pltpu.async_copy(gbuf, acc_sh.at[sidx], sems, add=True).wait()
        return carry

    lax.fori_loop(0, NCH, _chunk, 0)

    plsc.subcore_barrier()
    pltpu.sync_copy(acc_sh.at[pl.ds(s * RPW, RPW)],
                    out_hbm.at[c, pl.ds(s * RPW, RPW)])

    @pl.when(s == NS - 1)
    def _out_tail():
        pltpu.sync_copy(acc_sh.at[pl.ds(NS * RPW, TAIL)],
                        out_hbm.at[c, pl.ds(NS * RPW, TAIL)])


@functools.lru_cache(maxsize=None)
def _spmm_kernel():
    mesh = plsc.VectorSubcoreMesh(core_axis_name="c", subcore_axis_name="s",
                                  num_cores=NC, num_subcores=NS)
    return pl.kernel(
        _spmm_body,
        out_type=jax.ShapeDtypeStruct((NC, N, D), _f32),
        mesh=mesh,
        scratch_types=[
            pltpu.VMEM((NCH, K), _i32),
            pltpu.VMEM((K,), _i32),
            pltpu.VMEM((K,), _i32),
            pltpu.VMEM((K, 16), _f32),
            pltpu.VMEM((K, D), _f32),
            pltpu.VMEM_SHARED((N, D), _f32),
            pltpu.SemaphoreType.DMA,
            pltpu.SemaphoreType.DMA,
        ],
    )


def _elu(x):
    return jnp.where(x > 0, x, jnp.exp(jnp.minimum(x, 0.0)) - 1.0)


def _bn(x, gamma, beta):
    mu = jnp.mean(x, axis=0, keepdims=True)
    var = jnp.mean(x * x, axis=0, keepdims=True) - mu * mu
    return (x - mu) * lax.rsqrt(var + 1e-5) * gamma + beta


def _dot(a, b):
    return lax.dot_general(a, b, (((1,), (0,)), ((), ())),
                           precision=lax.Precision.HIGHEST,
                           preferred_element_type=_f32)


def _init_body(inp_ref, w_ref, b_ref, x_ref, h_ref):
    x = _dot(inp_ref[...], w_ref[...]) + b_ref[...]
    x_ref[...] = x
    h_ref[...] = _elu(x)


def _half_spmm_body(h_ref, p_ref, g_ref, be_ref, w_ref, b_ref,
                    *extra, do_elu, with_res):
    res_ref = extra[0] if with_res else None
    out_ref = extra[-1]
    h = h_ref[...]
    sp = p_ref[0] + p_ref[1]
    hn = _bn(h, g_ref[:, :D], be_ref[:, :D])
    sn = _bn(sp, g_ref[:, D:], be_ref[:, D:])
    y = _dot(hn, w_ref[:D]) + _dot(sn, w_ref[D:]) + b_ref[...]
    if with_res:
        y = y + res_ref[...]
    if do_elu:
        y = _elu(y)
    out_ref[...] = y


def _avg_layer_body(x_ref, g0, be0, w0, b0, g1, be1, w1, b1, x_out, h_out):
    x = x_ref[...]
    h = _elu(x)
    hn = _bn(h, g0[:, :D], be0[:, :D])
    h1 = _elu(_dot(hn, w0[:D]) + _dot(be0[:, D:], w0[D:]) + b0[...])
    h1n = _bn(h1, g1[:, :D], be1[:, :D])
    x2 = _dot(h1n, w1[:D]) + _dot(be1[:, D:], w1[D:]) + b1[...] + x
    x_out[...] = x2
    h_out[...] = _elu(x2)


def _avg_final_body(x_ref, g0, be0, w0, b0, g1, be1, w1, b1,
                    g2_ref, be2_ref, w2_ref, b2_ref, tiled_ref, out_ref):
    x = x_ref[...]
    h = _elu(x)
    hn = _bn(h, g0[:, :D], be0[:, :D])
    h1 = _elu(_dot(hn, w0[:D]) + _dot(be0[:, D:], w0[D:]) + b0[...])
    h1n = _bn(h1, g1[:, :D], be1[:, :D])
    x2 = _dot(h1n, w1[:D]) + _dot(be1[:, D:], w1[D:]) + b1[...] + x
    y = _elu(x2)
    yn = _bn(y, g2_ref[...], be2_ref[...])
    out_ref[...] = _dot(yn, w2_ref[...]) + b2_ref[...] + tiled_ref[...]


def _tc(body, out_shape, *args, **static):
    return pl.pallas_call(
        functools.partial(body, **static),
        out_shape=out_shape,
        compiler_params=pltpu.CompilerParams(
            vmem_limit_bytes=100 * 1024 * 1024),
    )(*args)


def kernel(L_indices, L_values, mask, inputs, W1, b1, rn_gamma, rn_beta,
           rn_W, rn_b, g2, be2, W2, b2):
    del mask
    inp2 = inputs[0]

    packed = jnp.pad(
        (L_indices[0].astype(_i32) << SH) | L_indices[1].astype(_i32),
        (0, E_PAD - E)).reshape(NW, NCH, K)
    vals = jnp.broadcast_to(
        jnp.pad(L_values, (0, E_PAD - E)).reshape(NW, NCH, K)[..., None],
        (NW, NCH, K, 16))

    inp_pad = jnp.pad(inp2, ((0, 0), (0, D - 3)))
    w1_pad = jnp.pad(W1, ((0, D - 3), (0, 0)))
    tiled = jnp.tile(inp2, (1, 40))

    nd = jax.ShapeDtypeStruct((N, D), _f32)
    spmm = _spmm_kernel()

    x, h = _tc(_init_body, (nd, nd), inp_pad, w1_pad, b1[None])

    for i in range(LAYERS):
        ga0, ga1 = rn_gamma[i, 0][None], rn_gamma[i, 1][None]
        be0, be1 = rn_beta[i, 0][None], rn_beta[i, 1][None]
        w0, w1 = rn_W[i, 0], rn_W[i, 1]
        bb0, bb1 = rn_b[i, 0][None], rn_b[i, 1][None]
        if i % 2 == 0:
            p = spmm(h, packed, vals)
            h1 = _tc(_half_spmm_body, nd, h, p, ga0, be0, w0, bb0,
                     do_elu=True, with_res=False)
            p2 = spmm(h1, packed, vals)
            x = _tc(_half_spmm_body, nd, h1, p2, ga1, be1, w1, bb1, x,
                    do_elu=False, with_res=True)
        elif i < LAYERS - 1:
            x, h = _tc(_avg_layer_body, (nd, nd), x,
                       ga0, be0, w0, bb0, ga1, be1, w1, bb1)
        else:
            out = _tc(_avg_final_body, jax.ShapeDtypeStruct((N, 120), _f32),
                      x, ga0, be0, w0, bb0, ga1, be1, w1, bb1,
                      g2[None], be2[None], W2, b2[None], tiled)
    return out[None]


# speedup vs baseline: 2.0606x; 1.0053x over previous
import functools

import jax
import jax.numpy as jnp
from jax import lax
from jax.experimental import pallas as pl
from jax.experimental.pallas import tpu as pltpu
from jax.experimental.pallas import tpu_sc as plsc

N = 10000
D = 128
E = 320000
LAYERS = 6

NC = 2
NS = 16
NW = NC * NS
K = 128
NCH = 80
EPW = K * NCH
E_PAD = NW * EPW
RPW = 624
TAIL = N - NS * RPW
SH = 14
MSK = (1 << SH) - 1

_f32 = jnp.float32
_i32 = jnp.int32


def _spmm_body(h_hbm, pk_hbm, vals_hbm, out_hbm,
               pk_v, gidx, sidx, vbuf, gbuf, acc_sh, semg, sems):
    c = lax.axis_index("c")
    s = lax.axis_index("s")
    wid = s * NC + c

    @plsc.parallel_loop(0, K * D // 16, unroll=8)
    def _zr(i):
        gbuf[i // 8, pl.ds((i % 8) * 16, 16)] = jnp.zeros((16,), _f32)
    for z in range(0, RPW, K):
        cnt = min(K, RPW - z)
        pltpu.sync_copy(gbuf.at[pl.ds(0, cnt)], acc_sh.at[pl.ds(s * RPW + z, cnt)])

    @pl.when(s == NS - 1)
    def _zero_tail():
        pltpu.sync_copy(gbuf.at[pl.ds(0, TAIL)], acc_sh.at[pl.ds(NS * RPW, TAIL)])

    plsc.subcore_barrier()

    pltpu.sync_copy(pk_hbm.at[wid], pk_v)

    def _chunk(j, carry):
        pltpu.sync_copy(vals_hbm.at[wid, j], vbuf)
        for g in range(K // 16):
            sl = pl.ds(g * 16, 16)
            pk = pk_v[j, sl]
            gidx[sl] = pk & MSK
            sidx[sl] = lax.shift_right_logical(pk, SH)
        pltpu.async_copy(h_hbm.at[gidx], gbuf, semg).wait()

        @plsc.parallel_loop(0, K, unroll=4)
        def _scale(e):
            vv = vbuf[e]
            for dd in range(D // 16):
                sl = pl.ds(dd * 16, 16)
                gbuf[e, sl] = gbuf[e, sl] * vv
        pltpu.async_copy(gbuf, acc_sh.at[sidx], sems, add=True).wait()
        return carry

    lax.fori_loop(0, NCH, _chunk, 0)

    plsc.subcore_barrier()
    pltpu.sync_copy(acc_sh.at[pl.ds(s * RPW, RPW)],
                    out_hbm.at[c, pl.ds(s * RPW, RPW)])

    @pl.when(s == NS - 1)
    def _out_tail():
        pltpu.sync_copy(acc_sh.at[pl.ds(NS * RPW, TAIL)],
                        out_hbm.at[c, pl.ds(NS * RPW, TAIL)])


@functools.lru_cache(maxsize=None)
def _spmm_kernel():
    mesh = plsc.VectorSubcoreMesh(core_axis_name="c", subcore_axis_name="s",
                                  num_cores=NC, num_subcores=NS)
    return pl.kernel(
        _spmm_body,
        out_type=jax.ShapeDtypeStruct((NC, N, D), _f32),
        mesh=mesh,
        scratch_types=[
            pltpu.VMEM((NCH, K), _i32),
            pltpu.VMEM((K,), _i32),
            pltpu.VMEM((K,), _i32),
            pltpu.VMEM((K, 16), _f32),
            pltpu.VMEM((K, D), _f32),
            pltpu.VMEM_SHARED((N, D), _f32),
            pltpu.SemaphoreType.DMA,
            pltpu.SemaphoreType.DMA,
        ],
    )


def _elu(x):
    return jnp.where(x > 0, x, jnp.exp(jnp.minimum(x, 0.0)) - 1.0)


def _bn(x, gamma, beta):
    mu = jnp.mean(x, axis=0, keepdims=True)
    var = jnp.mean(x * x, axis=0, keepdims=True) - mu * mu
    return (x - mu) * lax.rsqrt(var + 1e-5) * gamma + beta


def _dot(a, b):
    return lax.dot_general(a, b, (((1,), (0,)), ((), ())),
                           precision=lax.Precision.HIGHEST,
                           preferred_element_type=_f32)


def _init_body(inp_ref, w_ref, b_ref, x_ref, h_ref):
    x = _dot(inp_ref[...], w_ref[...]) + b_ref[...]
    x_ref[...] = x
    h_ref[...] = _elu(x)


def _half_spmm_body(h_ref, p_ref, g_ref, be_ref, w_ref, b_ref,
                    *extra, do_elu, with_res):
    res_ref = extra[0] if with_res else None
    out_ref = extra[-1]
    h = h_ref[...]
    sp = p_ref[0] + p_ref[1]
    hn = _bn(h, g_ref[:, :D], be_ref[:, :D])
    sn = _bn(sp, g_ref[:, D:], be_ref[:, D:])
    y = _dot(hn, w_ref[:D]) + _dot(sn, w_ref[D:]) + b_ref[...]
    if with_res:
        y = y + res_ref[...]
    if do_elu:
        y = _elu(y)
    out_ref[...] = y


def _avg_layer_body(x_ref, g0, be0, w0, b0, g1, be1, w1, b1, x_out, h_out):
    x = x_ref[...]
    h = _elu(x)
    hn = _bn(h, g0[:, :D], be0[:, :D])
    h1 = _elu(_dot(hn, w0[:D]) + _dot(be0[:, D:], w0[D:]) + b0[...])
    h1n = _bn(h1, g1[:, :D], be1[:, :D])
    x2 = _dot(h1n, w1[:D]) + _dot(be1[:, D:], w1[D:]) + b1[...] + x
    x_out[...] = x2
    h_out[...] = _elu(x2)


def _avg_final_body(x_ref, g0, be0, w0, b0, g1, be1, w1, b1,
                    g2_ref, be2_ref, w2_ref, b2_ref, tiled_ref, out_ref):
    x = x_ref[...]
    h = _elu(x)
    hn = _bn(h, g0[:, :D], be0[:, :D])
    h1 = _elu(_dot(hn, w0[:D]) + _dot(be0[:, D:], w0[D:]) + b0[...])
    h1n = _bn(h1, g1[:, :D], be1[:, :D])
    x2 = _dot(h1n, w1[:D]) + _dot(be1[:, D:], w1[D:]) + b1[...] + x
    y = _elu(x2)
    yn = _bn(y, g2_ref[...], be2_ref[...])
    out_ref[...] = _dot(yn, w2_ref[...]) + b2_ref[...] + tiled_ref[...]


def _tc(body, out_shape, *args, **static):
    return pl.pallas_call(
        functools.partial(body, **static),
        out_shape=out_shape,
        compiler_params=pltpu.CompilerParams(
            vmem_limit_bytes=100 * 1024 * 1024),
    )(*args)


def kernel(L_indices, L_values, mask, inputs, W1, b1, rn_gamma, rn_beta,
           rn_W, rn_b, g2, be2, W2, b2):
    del mask
    inp2 = inputs[0]

    packed = jnp.pad(
        (L_indices[0].astype(_i32) << SH) | L_indices[1].astype(_i32),
        (0, E_PAD - E)).reshape(NW, NCH, K)
    vals = jnp.broadcast_to(
        jnp.pad(L_values, (0, E_PAD - E)).reshape(NW, NCH, K)[..., None],
        (NW, NCH, K, 16))

    inp_pad = jnp.pad(inp2, ((0, 0), (0, D - 3)))
    w1_pad = jnp.pad(W1, ((0, D - 3), (0, 0)))
    tiled = jnp.tile(inp2, (1, 40))

    nd = jax.ShapeDtypeStruct((N, D), _f32)
    spmm = _spmm_kernel()

    x, h = _tc(_init_body, (nd, nd), inp_pad, w1_pad, b1[None])

    for i in range(LAYERS):
        ga0, ga1 = rn_gamma[i, 0][None], rn_gamma[i, 1][None]
        be0, be1 = rn_beta[i, 0][None], rn_beta[i, 1][None]
        w0, w1 = rn_W[i, 0], rn_W[i, 1]
        bb0, bb1 = rn_b[i, 0][None], rn_b[i, 1][None]
        if i % 2 == 0:
            p = spmm(h, packed, vals)
            h1 = _tc(_half_spmm_body, nd, h, p, ga0, be0, w0, bb0,
                     do_elu=True, with_res=False)
            p2 = spmm(h1, packed, vals)
            x = _tc(_half_spmm_body, nd, h1, p2, ga1, be1, w1, bb1, x,
                    do_elu=False, with_res=True)
        elif i < LAYERS - 1:
            x, h = _tc(_avg_layer_body, (nd, nd), x,
                       ga0, be0, w0, bb0, ga1, be1, w1, bb1)
        else:
            out = _tc(_avg_final_body, jax.ShapeDtypeStruct((N, 120), _f32),
                      x, ga0, be0, w0, bb0, ga1, be1, w1, bb1,
                      g2[None], be2[None], W2, b2[None], tiled)
    return out[None]


# speedup vs baseline: 2.1066x; 1.0223x over previous
import functools

import jax
import jax.numpy as jnp
from jax import lax
from jax.experimental import pallas as pl
from jax.experimental.pallas import tpu as pltpu
from jax.experimental.pallas import tpu_sc as plsc

N = 10000
D = 128
E = 320000
LAYERS = 6

NC = 2
NS = 16
NW = NC * NS
K = 64
NCH = 160
EPW = K * NCH
E_PAD = NW * EPW
RPW = 624
TAIL = N - NS * RPW
SH = 14
MSK = (1 << SH) - 1

_f32 = jnp.float32
_i32 = jnp.int32


def _spmm_body(h_hbm, pk_hbm, vals_hbm, out_hbm,
               pk_v, gidx0, sidx0, gidx1, sidx1, vbuf0, vbuf1,
               gbuf0, gbuf1, acc_sh, semg0, semg1, sems0, sems1):
    c = lax.axis_index("c")
    s = lax.axis_index("s")
    wid = s * NC + c

    @plsc.parallel_loop(0, K * D // 16, unroll=8)
    def _zr(i):
        gbuf0[i // 8, pl.ds((i % 8) * 16, 16)] = jnp.zeros((16,), _f32)
    for z in range(0, RPW, K):
        cnt = min(K, RPW - z)
        pltpu.sync_copy(gbuf0.at[pl.ds(0, cnt)], acc_sh.at[pl.ds(s * RPW + z, cnt)])

    @pl.when(s == NS - 1)
    def _zero_tail():
        pltpu.sync_copy(gbuf0.at[pl.ds(0, TAIL)], acc_sh.at[pl.ds(NS * RPW, TAIL)])

    plsc.subcore_barrier()

    def _idx(j, gidx, sidx):
        for g in range(K // 16):
            sl = pl.ds(g * 16, 16)
            pk = pk_v[j, sl]
            gidx[sl] = pk & MSK
            sidx[sl] = lax.shift_right_logical(pk, SH)

    def _scale(vbuf, gbuf):
        @plsc.parallel_loop(0, K, unroll=4)
        def _s(e):
            vv = vbuf[e]
            for dd in range(D // 16):
                sl = pl.ds(dd * 16, 16)
                gbuf[e, sl] = gbuf[e, sl] * vv

    def _pair(t, carry):
        j0 = t * 2
        pltpu.sync_copy(pk_hbm.at[wid, pl.ds(j0, 2)], pk_v)
        pltpu.sync_copy(vals_hbm.at[wid, j0], vbuf0)
        pltpu.sync_copy(vals_hbm.at[wid, j0 + 1], vbuf1)
        _idx(0, gidx0, sidx0)
        cg0 = pltpu.async_copy(h_hbm.at[gidx0], gbuf0, semg0)
        _idx(1, gidx1, sidx1)
        cg1 = pltpu.async_copy(h_hbm.at[gidx1], gbuf1, semg1)
        cg0.wait()
        _scale(vbuf0, gbuf0)
        cs0 = pltpu.async_copy(gbuf0, acc_sh.at[sidx0], sems0, add=True)
        cg1.wait()
        _scale(vbuf1, gbuf1)
        cs1 = pltpu.async_copy(gbuf1, acc_sh.at[sidx1], sems1, add=True)
        cs0.wait()
        cs1.wait()
        return carry

    lax.fori_loop(0, NCH // 2, _pair, 0)

    plsc.subcore_barrier()
    pltpu.sync_copy(acc_sh.at[pl.ds(s * RPW, RPW)],
                    out_hbm.at[c, pl.ds(s * RPW, RPW)])

    @pl.when(s == NS - 1)
    def _out_tail():
        pltpu.sync_copy(acc_sh.at[pl.ds(NS * RPW, TAIL)],
                        out_hbm.at[c, pl.ds(NS * RPW, TAIL)])


@functools.lru_cache(maxsize=None)
def _spmm_kernel():
    mesh = plsc.VectorSubcoreMesh(core_axis_name="c", subcore_axis_name="s",
                                  num_cores=NC, num_subcores=NS)
    return pl.kernel(
        _spmm_body,
        out_type=jax.ShapeDtypeStruct((NC, N, D), _f32),
        mesh=mesh,
        scratch_types=[
            pltpu.VMEM((2, K), _i32),
            pltpu.VMEM((K,), _i32),
            pltpu.VMEM((K,), _i32),
            pltpu.VMEM((K,), _i32),
            pltpu.VMEM((K,), _i32),
            pltpu.VMEM((K, 16), _f32),
            pltpu.VMEM((K, 16), _f32),
            pltpu.VMEM((K, D), _f32),
            pltpu.VMEM((K, D), _f32),
            pltpu.VMEM_SHARED((N, D), _f32),
            pltpu.SemaphoreType.DMA,
            pltpu.SemaphoreType.DMA,
            pltpu.SemaphoreType.DMA,
            pltpu.SemaphoreType.DMA,
        ],
    )


def _elu(x):
    return jnp.where(x > 0, x, jnp.exp(jnp.minimum(x, 0.0)) - 1.0)


def _bn(x, gamma, beta):
    mu = jnp.mean(x, axis=0, keepdims=True)
    var = jnp.mean(x * x, axis=0, keepdims=True) - mu * mu
    return (x - mu) * lax.rsqrt(var + 1e-5) * gamma + beta


def _dot(a, b):
    return lax.dot_general(a, b, (((1,), (0,)), ((), ())),
                           precision=lax.Precision.HIGHEST,
                           preferred_element_type=_f32)


def _init_body(inp_ref, w_ref, b_ref, x_ref, h_ref):
    x = _dot(inp_ref[...], w_ref[...]) + b_ref[...]
    x_ref[...] = x
    h_ref[...] = _elu(x)


def _half_spmm_body(h_ref, p_ref, g_ref, be_ref, w_ref, b_ref,
                    *extra, do_elu, with_res):
    res_ref = extra[0] if with_res else None
    out_ref = extra[-1]
    h = h_ref[...]
    sp = p_ref[0] + p_ref[1]
    hn = _bn(h, g_ref[:, :D], be_ref[:, :D])
    sn = _bn(sp, g_ref[:, D:], be_ref[:, D:])
    y = _dot(hn, w_ref[:D]) + _dot(sn, w_ref[D:]) + b_ref[...]
    if with_res:
        y = y + res_ref[...]
    if do_elu:
        y = _elu(y)
    out_ref[...] = y


def _avg_layer_body(x_ref, g0, be0, w0, b0, g1, be1, w1, b1, x_out, h_out):
    x = x_ref[...]
    h = _elu(x)
    hn = _bn(h, g0[:, :D], be0[:, :D])
    h1 = _elu(_dot(hn, w0[:D]) + _dot(be0[:, D:], w0[D:]) + b0[...])
    h1n = _bn(h1, g1[:, :D], be1[:, :D])
    x2 = _dot(h1n, w1[:D]) + _dot(be1[:, D:], w1[D:]) + b1[...] + x
    x_out[...] = x2
    h_out[...] = _elu(x2)


def _avg_final_body(x_ref, g0, be0, w0, b0, g1, be1, w1, b1,
                    g2_ref, be2_ref, w2_ref, b2_ref, tiled_ref, out_ref):
    x = x_ref[...]
    h = _elu(x)
    hn = _bn(h, g0[:, :D], be0[:, :D])
    h1 = _elu(_dot(hn, w0[:D]) + _dot(be0[:, D:], w0[D:]) + b0[...])
    h1n = _bn(h1, g1[:, :D], be1[:, :D])
    x2 = _dot(h1n, w1[:D]) + _dot(be1[:, D:], w1[D:]) + b1[...] + x
    y = _elu(x2)
    yn = _bn(y, g2_ref[...], be2_ref[...])
    out_ref[...] = _dot(yn, w2_ref[...]) + b2_ref[...] + tiled_ref[...]


def _tc(body, out_shape, *args, **static):
    return pl.pallas_call(
        functools.partial(body, **static),
        out_shape=out_shape,
        compiler_params=pltpu.CompilerParams(
            vmem_limit_bytes=100 * 1024 * 1024),
    )(*args)


def kernel(L_indices, L_values, mask, inputs, W1, b1, rn_gamma, rn_beta,
           rn_W, rn_b, g2, be2, W2, b2):
    del mask
    inp2 = inputs[0]

    packed = jnp.pad(
        (L_indices[0].astype(_i32) << SH) | L_indices[1].astype(_i32),
        (0, E_PAD - E)).reshape(NW, NCH, K)
    vals = jnp.broadcast_to(
        jnp.pad(L_values, (0, E_PAD - E)).reshape(NW, NCH, K)[..., None],
        (NW, NCH, K, 16))

    inp_pad = jnp.pad(inp2, ((0, 0), (0, D - 3)))
    w1_pad = jnp.pad(W1, ((0, D - 3), (0, 0)))
    tiled = jnp.tile(inp2, (1, 40))

    nd = jax.ShapeDtypeStruct((N, D), _f32)
    spmm = _spmm_kernel()

    x, h = _tc(_init_body, (nd, nd), inp_pad, w1_pad, b1[None])

    for i in range(LAYERS):
        ga0, ga1 = rn_gamma[i, 0][None], rn_gamma[i, 1][None]
        be0, be1 = rn_beta[i, 0][None], rn_beta[i, 1][None]
        w0, w1 = rn_W[i, 0], rn_W[i, 1]
        bb0, bb1 = rn_b[i, 0][None], rn_b[i, 1][None]
        if i % 2 == 0:
            p = spmm(h, packed, vals)
            h1 = _tc(_half_spmm_body, nd, h, p, ga0, be0, w0, bb0,
                     do_elu=True, with_res=False)
            p2 = spmm(h1, packed, vals)
            x = _tc(_half_spmm_body, nd, h1, p2, ga1, be1, w1, bb1, x,
                    do_elu=False, with_res=True)
        elif i < LAYERS - 1:
            x, h = _tc(_avg_layer_body, (nd, nd), x,
                       ga0, be0, w0, bb0, ga1, be1, w1, bb1)
        else:
            out = _tc(_avg_final_body, jax.ShapeDtypeStruct((N, 120), _f32),
                      x, ga0, be0, w0, bb0, ga1, be1, w1, bb1,
                      g2[None], be2[None], W2, b2[None], tiled)
    return out[None]
